# baseline (speedup 1.0000x reference)
import functools

import jax
import jax.numpy as jnp
from jax import lax
from jax.experimental import pallas as pl
from jax.experimental.pallas import tpu as pltpu
from jax.experimental.pallas import tpu_sc as plsc

_BATCH = 16384
_DIM = 32
_K = 3 * _BATCH
_NW = 32
_BPW = _BATCH // _NW
_WL = 512
_BLKS_PER_TILE = 61
_NWIN = 62
_TAIL_BASE = 999936
_CAP = 8192
_STAGE = 1024
_SCRATCH = _DIM * _K
_DUMP = _SCRATCH
_SCRATCH_PAD = _SCRATCH + 16

_mesh = plsc.VectorSubcoreMesh(core_axis_name="c", subcore_axis_name="s")
_IOTA = lambda: lax.iota(jnp.int32, 16)


@functools.partial(
    pl.kernel,
    mesh=_mesh,
    out_type=jax.ShapeDtypeStruct((_SCRATCH_PAD,), jnp.float32),
    compiler_params=pltpu.CompilerParams(
        needs_layout_passes=False, use_tc_tiling_on_sc=True),
    scratch_types=[
        pltpu.VMEM((1024,), jnp.int32),
        pltpu.VMEM((1024,), jnp.int32),
        pltpu.VMEM((_CAP + 16,), jnp.int32),
        pltpu.VMEM((_CAP + 16,), jnp.int32),
        pltpu.VMEM((_CAP + 16,), jnp.int32),
        pltpu.VMEM((_CAP + 16,), jnp.int32),
        pltpu.VMEM((_DIM, _WL), jnp.float32),
        pltpu.VMEM((_DIM, _WL), jnp.float32),
        pltpu.VMEM((_DIM, _WL), jnp.float32),
        pltpu.VMEM((_DIM, _WL), jnp.float32),
        pltpu.VMEM((_DIM, 64), jnp.float32),
        pltpu.VMEM((_DIM, 64), jnp.float32),
        pltpu.VMEM((16,), jnp.int32),
        pltpu.VMEM((16,), jnp.int32),
        pltpu.VMEM((_STAGE,), jnp.float32),
        pltpu.VMEM((_STAGE,), jnp.int32),
        pltpu.SemaphoreType.DMA,
        pltpu.SemaphoreType.DMA,
        pltpu.SemaphoreType.DMA,
        pltpu.SemaphoreType.DMA,
        pltpu.SemaphoreType.DMA,
    ],
)
def _harvest(tuples_hbm, nodeT_hbm, relaT_hbm, scratch_hbm,
             cbuf0, cbuf1, hNi, hNk, hRi, hRk,
             nwin0, nwin1, rwin0, rwin1, ntail, rtail,
             clane, cslot, stv, std, semi0, semi1, semw0, semw1, semf):
    wid = lax.axis_index("s") * 2 + lax.axis_index("c")
    blk0 = wid * _BLKS_PER_TILE
    lane_lo = blk0 * _WL
    is_last = wid == _NW - 1
    lane_hi = jnp.where(is_last, 1000000, lane_lo + _BLKS_PER_TILE * _WL)
    iota = _IOTA()
    dump_vec = jnp.full((16,), _DUMP, jnp.int32)

    for s in range(_STAGE // 16):
        std[pl.ds(s * 16, 16)] = dump_vec

    cbufs = (cbuf0, cbuf1)
    nwins = (nwin0, nwin1)
    rwins = (rwin0, rwin1)
    semis = (semi0, semi1)
    semws = (semw0, semw1)
    NCHUNK = _K // 1024

    def scan_round(rlo):
        pltpu.async_copy(tuples_hbm.at[pl.ds(0, 1024)], cbufs[0], semis[0])
        pltpu.async_copy(tuples_hbm.at[pl.ds(1024, 1024)], cbufs[1], semis[1])

        def pair_body(p, carry):
            cN, cR = carry
            for b in range(2):
                c = 2 * p + b
                pltpu.make_async_copy(
                    tuples_hbm.at[pl.ds(0, 1024)], cbufs[b], semis[b]).wait()

                def vreg_body(v, carry2, b=b):
                    cN2, cR2 = carry2
                    iv = cbufs[b][pl.ds(v * 16, 16)]
                    kvec = c * 1024 + v * 16 + iota
                    inr = (iv >= lane_lo) & (iv < lane_hi)
                    mN = inr & (kvec < 2 * _BATCH)
                    mR = inr & (kvec >= 2 * _BATCH)

                    def emit(mask, cnt, hi_ref, hk_ref):
                        mi = mask.astype(jnp.int32)
                        csum = lax.cumsum(mi, axis=0)
                        rank = cnt + csum - 1
                        ok = mask & (rank >= rlo) & (rank < rlo + _CAP)
                        pos = jnp.clip(cnt - rlo, 0, _CAP)
                        plsc.store_compressed(hi_ref.at[pl.ds(pos, 16)], iv,
                                              mask=ok)
                        plsc.store_compressed(hk_ref.at[pl.ds(pos, 16)],
                                              kvec, mask=ok)
                        return cnt + csum[15]

                    cN2 = emit(mN, cN2, hNi, hNk)
                    cR2 = emit(mR, cR2, hRi, hRk)
                    return cN2, cR2

                cN, cR = lax.fori_loop(0, 64, vreg_body, (cN, cR))

                def refire(c=c, b=b):
                    off = pl.multiple_of((c + 2) * 1024, 1024)
                    pltpu.async_copy(
                        tuples_hbm.at[pl.ds(off, 1024)], cbufs[b], semis[b])

                pl.when(c + 2 < NCHUNK)(refire)
            return cN, cR

        return lax.fori_loop(0, NCHUNK // 2, pair_body,
                             (jnp.int32(0), jnp.int32(0)))

    def harvest_win(buf, width, wbase, nhits, hi_ref, hk_ref, c2):
        nv = (nhits + 15) // 16

        def vreg_body(v, c2):
            hv = hi_ref[pl.ds(v * 16, 16)]
            hk = hk_ref[pl.ds(v * 16, 16)]
            lanev = hv - wbase
            inwin = (hv >= wbase) & (hv < wbase + width)
            inwin = inwin & ((v * 16 + iota) < nhits)

            def do_vreg(c2):
                m = jnp.sum(inwin.astype(jnp.int32))
                plsc.store_compressed(clane.at[pl.ds(0, 16)], lanev,
                                      mask=inwin)
                plsc.store_compressed(cslot.at[pl.ds(0, 16)], hk, mask=inwin)

                def hit_body(j, c2):
                    jv = jnp.full((16,), j, jnp.int32)
                    colv = plsc.load_gather(clane, [jv])
                    kv = plsc.load_gather(cslot, [jv])
                    rows_lo = iota
                    rows_hi = iota + 16
                    v_lo = plsc.load_gather(buf, [rows_lo, colv])
                    v_hi = plsc.load_gather(buf, [rows_hi, colv])
                    d_lo = iota * _K + kv
                    d_hi = (iota + 16) * _K + kv
                    stv[pl.ds(c2, 16)] = v_lo
                    std[pl.ds(c2, 16)] = d_lo
                    stv[pl.ds(c2 + 16, 16)] = v_hi
                    std[pl.ds(c2 + 16, 16)] = d_hi
                    c2n = c2 + 32

                    def flush():
                        pltpu.async_copy(
                            stv, scratch_hbm.at[std], semf).wait()
                        for s in range(_STAGE // 16):
                            std[pl.ds(s * 16, 16)] = dump_vec

                    do_flush = c2n > _STAGE - 32
                    pl.when(do_flush)(flush)
                    return jnp.where(do_flush, 0, c2n)

                return lax.fori_loop(0, m, hit_body, c2)

            any_hit = jnp.any(inwin)
            return lax.cond(any_hit, do_vreg, lambda c2: c2, c2)

        return lax.fori_loop(0, nv, vreg_body, c2)

    def round_body(carry):
        r, c2, doneN, doneR, totN, totR = carry
        rlo = r * _CAP
        totN, totR = scan_round(rlo)
        nN = jnp.clip(totN - rlo, 0, _CAP)
        nR = jnp.clip(totR - rlo, 0, _CAP)

        off0 = pl.multiple_of(blk0 * _WL, _WL)
        off1 = pl.multiple_of((blk0 + 1) * _WL, _WL)
        pltpu.async_copy(nodeT_hbm.at[:, pl.ds(off0, _WL)], nwins[0], semws[0])
        pltpu.async_copy(relaT_hbm.at[:, pl.ds(off0, _WL)], rwins[0], semws[0])
        pltpu.async_copy(nodeT_hbm.at[:, pl.ds(off1, _WL)], nwins[1], semws[1])
        pltpu.async_copy(relaT_hbm.at[:, pl.ds(off1, _WL)], rwins[1], semws[1])

        def win_pair(p, c2):
            for b in range(2):
                w = 2 * p + b
                pltpu.make_async_copy(
                    nodeT_hbm.at[:, pl.ds(0, _WL)], nwins[b],
                    semws[b]).wait()
                pltpu.make_async_copy(
                    relaT_hbm.at[:, pl.ds(0, _WL)], rwins[b],
                    semws[b]).wait()
                wbase = (blk0 + w) * _WL
                c2 = harvest_win(nwins[b], _WL, wbase, nN, hNi, hNk, c2)
                c2 = harvest_win(rwins[b], _WL, wbase, nR, hRi, hRk, c2)

                def refire(w=w, b=b):
                    off = pl.multiple_of((blk0 + w + 2) * _WL, _WL)
                    pltpu.async_copy(nodeT_hbm.at[:, pl.ds(off, _WL)],
                                     nwins[b], semws[b])
                    pltpu.async_copy(relaT_hbm.at[:, pl.ds(off, _WL)],
                                     rwins[b], semws[b])

                pl.when(w + 2 < _NWIN)(refire)
            return c2

        c2 = lax.fori_loop(0, _NWIN // 2, win_pair, c2)

        def tail(c2):
            pltpu.async_copy(
                nodeT_hbm.at[:, pl.ds(_TAIL_BASE, 64)], ntail, semws[0]).wait()
            pltpu.async_copy(
                relaT_hbm.at[:, pl.ds(_TAIL_BASE, 64)], rtail, semws[0]).wait()
            c2 = harvest_win(ntail, 64, _TAIL_BASE, nN, hNi, hNk, c2)
            c2 = harvest_win(rtail, 64, _TAIL_BASE, nR, hRi, hRk, c2)
            return c2

        c2 = lax.cond(is_last, tail, lambda c2: c2, c2)
        doneN = jnp.minimum(rlo + _CAP, totN)
        doneR = jnp.minimum(rlo + _CAP, totR)
        return r + 1, c2, doneN, doneR, totN, totR

    def round_cond(carry):
        r, c2, doneN, doneR, totN, totR = carry
        return (r == 0) | (doneN < totN) | (doneR < totR)

    init = (jnp.int32(0), jnp.int32(0), jnp.int32(0), jnp.int32(0),
            jnp.int32(-1), jnp.int32(-1))
    _, c2, _, _, _, _ = lax.while_loop(round_cond, round_body, init)

    def final_flush():
        pltpu.async_copy(stv, scratch_hbm.at[std], semf).wait()

    pl.when(c2 > 0)(final_flush)


@functools.partial(
    pl.kernel,
    mesh=_mesh,
    out_type=jax.ShapeDtypeStruct((_BATCH,), jnp.float32),
    compiler_params=pltpu.CompilerParams(
        needs_layout_passes=False, use_tc_tiling_on_sc=True),
    scratch_types=[
        pltpu.VMEM((_DIM, _BPW), jnp.float32),
        pltpu.VMEM((_DIM, _BPW), jnp.float32),
        pltpu.VMEM((_DIM, _BPW), jnp.float32),
        pltpu.VMEM((_BPW,), jnp.float32),
        pltpu.SemaphoreType.DMA,
    ],
)
def _reduce(scratch_hbm, out_hbm, hbuf, tbuf, rbuf, outv, sem):
    wid = lax.axis_index("s") * 2 + lax.axis_index("c")
    base = wid * _BPW
    copies = []
    for d in range(_DIM):
        copies.append(pltpu.async_copy(
            scratch_hbm.at[pl.ds(d * _K + base, _BPW)], hbuf.at[d], sem))
        copies.append(pltpu.async_copy(
            scratch_hbm.at[pl.ds(d * _K + _BATCH + base, _BPW)],
            tbuf.at[d], sem))
        copies.append(pltpu.async_copy(
            scratch_hbm.at[pl.ds(d * _K + 2 * _BATCH + base, _BPW)],
            rbuf.at[d], sem))
    for c in copies:
        c.wait()

    def group_body(g, carry):
        s = pl.ds(g * 16, 16)
        acc = jnp.zeros((16,), jnp.float32)
        for d in range(_DIM):
            acc = acc + hbuf[d, s] * rbuf[d, s] * tbuf[d, s]
        outv[s] = acc
        return carry

    lax.fori_loop(0, _BPW // 16, group_body, 0)
    pltpu.sync_copy(outv, out_hbm.at[pl.ds(base, _BPW)])


def kernel(tuples, node_emb, rela_emb):
    scratch = _harvest(tuples.reshape(-1), node_emb.T, rela_emb.T)
    return _reduce(scratch)


# speedup vs baseline: 11.1235x; 11.1235x over previous
import functools

import jax
import jax.numpy as jnp
from jax import lax
from jax.experimental import pallas as pl
from jax.experimental.pallas import tpu as pltpu
from jax.experimental.pallas import tpu_sc as plsc

_BATCH = 16384
_DIM = 32
_K = 3 * _BATCH
_NW = 32
_BPW = _BATCH // _NW
_WL = 512
_BLKS_PER_TILE = 61
_NWIN = 62
_TAIL_BASE = 999936
_CAP = 8192
_STAGE = 1024
_SCRATCH = _DIM * _K
_DUMP = _SCRATCH
_SCRATCH_PAD = _SCRATCH + 16

_mesh = plsc.VectorSubcoreMesh(core_axis_name="c", subcore_axis_name="s")
_IOTA = lambda: lax.iota(jnp.int32, 16)


@functools.partial(
    pl.kernel,
    mesh=_mesh,
    out_type=jax.ShapeDtypeStruct((_SCRATCH_PAD,), jnp.float32),
    compiler_params=pltpu.CompilerParams(
        needs_layout_passes=False, use_tc_tiling_on_sc=True),
    scratch_types=[
        pltpu.VMEM((1024,), jnp.int32),
        pltpu.VMEM((1024,), jnp.int32),
        pltpu.VMEM((_CAP + 16,), jnp.int32),
        pltpu.VMEM((_CAP + 16,), jnp.int32),
        pltpu.VMEM((_CAP + 16,), jnp.int32),
        pltpu.VMEM((_CAP + 16,), jnp.int32),
        pltpu.VMEM((_DIM, _WL), jnp.float32),
        pltpu.VMEM((_DIM, _WL), jnp.float32),
        pltpu.VMEM((_DIM, _WL), jnp.float32),
        pltpu.VMEM((_DIM, _WL), jnp.float32),
        pltpu.VMEM((_DIM, 64), jnp.float32),
        pltpu.VMEM((_DIM, 64), jnp.float32),
        pltpu.VMEM((16,), jnp.int32),
        pltpu.VMEM((16,), jnp.int32),
        pltpu.VMEM((_STAGE,), jnp.float32),
        pltpu.VMEM((_STAGE,), jnp.int32),
        pltpu.SemaphoreType.DMA,
        pltpu.SemaphoreType.DMA,
        pltpu.SemaphoreType.DMA,
        pltpu.SemaphoreType.DMA,
        pltpu.SemaphoreType.DMA,
    ],
)
def _harvest(tuples_hbm, nodeT_hbm, relaT_hbm, scratch_hbm,
             cbuf0, cbuf1, hNi, hNk, hRi, hRk,
             nwin0, nwin1, rwin0, rwin1, ntail, rtail,
             clane, cslot, stv, std, semi0, semi1, semw0, semw1, semf):
    wid = lax.axis_index("s") * 2 + lax.axis_index("c")
    blk0 = wid * _BLKS_PER_TILE
    lane_lo = blk0 * _WL
    is_last = wid == _NW - 1
    lane_hi = jnp.where(is_last, 1000000, lane_lo + _BLKS_PER_TILE * _WL)
    iota = _IOTA()
    dump_vec = jnp.full((16,), _DUMP, jnp.int32)

    for s in range(_STAGE // 16):
        std[pl.ds(s * 16, 16)] = dump_vec

    cbufs = (cbuf0, cbuf1)
    nwins = (nwin0, nwin1)
    rwins = (rwin0, rwin1)
    semis = (semi0, semi1)
    semws = (semw0, semw1)
    NCHUNK = _K // 1024

    def scan_round(rlo):
        pltpu.async_copy(tuples_hbm.at[pl.ds(0, 1024)], cbufs[0], semis[0])
        pltpu.async_copy(tuples_hbm.at[pl.ds(1024, 1024)], cbufs[1], semis[1])

        def pair_body(p, carry):
            cN, cR = carry
            for b in range(2):
                c = 2 * p + b
                pltpu.make_async_copy(
                    tuples_hbm.at[pl.ds(0, 1024)], cbufs[b], semis[b]).wait()

                def vreg_body(v, carry2, b=b):
                    cN2, cR2 = carry2
                    iv = cbufs[b][pl.ds(v * 16, 16)]
                    kvec = c * 1024 + v * 16 + iota
                    inr = (iv >= lane_lo) & (iv < lane_hi)
                    mN = inr & (kvec < 2 * _BATCH)
                    mR = inr & (kvec >= 2 * _BATCH)

                    def emit(mask, cnt, hi_ref, hk_ref):
                        mi = mask.astype(jnp.int32)
                        csum = lax.cumsum(mi, axis=0)
                        rank = cnt + csum - 1
                        ok = mask & (rank >= rlo) & (rank < rlo + _CAP)
                        pos = jnp.clip(cnt - rlo, 0, _CAP)
                        plsc.store_compressed(hi_ref.at[pl.ds(pos, 16)], iv,
                                              mask=ok)
                        plsc.store_compressed(hk_ref.at[pl.ds(pos, 16)],
                                              kvec, mask=ok)
                        return cnt + csum[15]

                    cN2 = emit(mN, cN2, hNi, hNk)
                    cR2 = emit(mR, cR2, hRi, hRk)
                    return cN2, cR2

                cN, cR = lax.fori_loop(0, 64, vreg_body, (cN, cR))

                def refire(c=c, b=b):
                    off = pl.multiple_of((c + 2) * 1024, 1024)
                    pltpu.async_copy(
                        tuples_hbm.at[pl.ds(off, 1024)], cbufs[b], semis[b])

                pl.when(c + 2 < NCHUNK)(refire)
            return cN, cR

        return lax.fori_loop(0, NCHUNK // 2, pair_body,
                             (jnp.int32(0), jnp.int32(0)))

    def harvest_win(buf, width, wbase, nhits, hi_ref, hk_ref, c2):
        nv = (nhits + 15) // 16

        def vreg_body(v, c2):
            hv = hi_ref[pl.ds(v * 16, 16)]
            hk = hk_ref[pl.ds(v * 16, 16)]
            lanev = hv - wbase
            inwin = (hv >= wbase) & (hv < wbase + width)
            inwin = inwin & ((v * 16 + iota) < nhits)

            def do_vreg(c2):
                m = jnp.sum(inwin.astype(jnp.int32))
                plsc.store_compressed(clane.at[pl.ds(0, 16)], lanev,
                                      mask=inwin)
                plsc.store_compressed(cslot.at[pl.ds(0, 16)], hk, mask=inwin)

                def hit_body(j, c2):
                    jv = jnp.full((16,), j, jnp.int32)
                    colv = plsc.load_gather(clane, [jv])
                    kv = plsc.load_gather(cslot, [jv])
                    rows_lo = iota
                    rows_hi = iota + 16
                    v_lo = plsc.load_gather(buf, [rows_lo, colv])
                    v_hi = plsc.load_gather(buf, [rows_hi, colv])
                    d_lo = iota * _K + kv
                    d_hi = (iota + 16) * _K + kv
                    stv[pl.ds(c2, 16)] = v_lo
                    std[pl.ds(c2, 16)] = d_lo
                    stv[pl.ds(c2 + 16, 16)] = v_hi
                    std[pl.ds(c2 + 16, 16)] = d_hi
                    c2n = c2 + 32

                    def flush():
                        pltpu.async_copy(
                            stv, scratch_hbm.at[std], semf).wait()
                        for s in range(_STAGE // 16):
                            std[pl.ds(s * 16, 16)] = dump_vec

                    do_flush = c2n > _STAGE - 32
                    pl.when(do_flush)(flush)
                    return jnp.where(do_flush, 0, c2n)

                return lax.fori_loop(0, m, hit_body, c2)

            any_hit = jnp.any(inwin)
            return lax.cond(any_hit, do_vreg, lambda c2: c2, c2)

        return lax.fori_loop(0, nv, vreg_body, c2)

    def round_body(carry):
        r, c2, doneN, doneR, totN, totR = carry
        rlo = r * _CAP
        totN, totR = scan_round(rlo)
        nN = jnp.clip(totN - rlo, 0, _CAP)
        nR = jnp.clip(totR - rlo, 0, _CAP)

        off0 = pl.multiple_of(blk0 * _WL, _WL)
        off1 = pl.multiple_of((blk0 + 1) * _WL, _WL)
        pltpu.async_copy(nodeT_hbm.at[:, pl.ds(off0, _WL)], nwins[0], semws[0])
        pltpu.async_copy(relaT_hbm.at[:, pl.ds(off0, _WL)], rwins[0], semws[0])
        pltpu.async_copy(nodeT_hbm.at[:, pl.ds(off1, _WL)], nwins[1], semws[1])
        pltpu.async_copy(relaT_hbm.at[:, pl.ds(off1, _WL)], rwins[1], semws[1])

        def win_pair(p, c2):
            for b in range(2):
                w = 2 * p + b
                pltpu.make_async_copy(
                    nodeT_hbm.at[:, pl.ds(0, _WL)], nwins[b],
                    semws[b]).wait()
                pltpu.make_async_copy(
                    relaT_hbm.at[:, pl.ds(0, _WL)], rwins[b],
                    semws[b]).wait()
                wbase = (blk0 + w) * _WL
                pass

                def refire(w=w, b=b):
                    off = pl.multiple_of((blk0 + w + 2) * _WL, _WL)
                    pltpu.async_copy(nodeT_hbm.at[:, pl.ds(off, _WL)],
                                     nwins[b], semws[b])
                    pltpu.async_copy(relaT_hbm.at[:, pl.ds(off, _WL)],
                                     rwins[b], semws[b])

                pl.when(w + 2 < _NWIN)(refire)
            return c2

        c2 = lax.fori_loop(0, _NWIN // 2, win_pair, c2)

        def tail(c2):
            pltpu.async_copy(
                nodeT_hbm.at[:, pl.ds(_TAIL_BASE, 64)], ntail, semws[0]).wait()
            pltpu.async_copy(
                relaT_hbm.at[:, pl.ds(_TAIL_BASE, 64)], rtail, semws[0]).wait()
            c2 = harvest_win(ntail, 64, _TAIL_BASE, nN, hNi, hNk, c2)
            c2 = harvest_win(rtail, 64, _TAIL_BASE, nR, hRi, hRk, c2)
            return c2

        c2 = lax.cond(is_last, tail, lambda c2: c2, c2)
        doneN = jnp.minimum(rlo + _CAP, totN)
        doneR = jnp.minimum(rlo + _CAP, totR)
        return r + 1, c2, doneN, doneR, totN, totR

    def round_cond(carry):
        r, c2, doneN, doneR, totN, totR = carry
        return (r == 0) | (doneN < totN) | (doneR < totR)

    init = (jnp.int32(0), jnp.int32(0), jnp.int32(0), jnp.int32(0),
            jnp.int32(-1), jnp.int32(-1))
    _, c2, _, _, _, _ = lax.while_loop(round_cond, round_body, init)

    def final_flush():
        pltpu.async_copy(stv, scratch_hbm.at[std], semf).wait()

    pl.when(c2 > 0)(final_flush)


@functools.partial(
    pl.kernel,
    mesh=_mesh,
    out_type=jax.ShapeDtypeStruct((_BATCH,), jnp.float32),
    compiler_params=pltpu.CompilerParams(
        needs_layout_passes=False, use_tc_tiling_on_sc=True),
    scratch_types=[
        pltpu.VMEM((_DIM, _BPW), jnp.float32),
        pltpu.VMEM((_DIM, _BPW), jnp.float32),
        pltpu.VMEM((_DIM, _BPW), jnp.float32),
        pltpu.VMEM((_BPW,), jnp.float32),
        pltpu.SemaphoreType.DMA,
    ],
)
def _reduce(scratch_hbm, out_hbm, hbuf, tbuf, rbuf, outv, sem):
    wid = lax.axis_index("s") * 2 + lax.axis_index("c")
    base = wid * _BPW
    copies = []
    for d in range(_DIM):
        copies.append(pltpu.async_copy(
            scratch_hbm.at[pl.ds(d * _K + base, _BPW)], hbuf.at[d], sem))
        copies.append(pltpu.async_copy(
            scratch_hbm.at[pl.ds(d * _K + _BATCH + base, _BPW)],
            tbuf.at[d], sem))
        copies.append(pltpu.async_copy(
            scratch_hbm.at[pl.ds(d * _K + 2 * _BATCH + base, _BPW)],
            rbuf.at[d], sem))
    for c in copies:
        c.wait()

    def group_body(g, carry):
        s = pl.ds(g * 16, 16)
        acc = jnp.zeros((16,), jnp.float32)
        for d in range(_DIM):
            acc = acc + hbuf[d, s] * rbuf[d, s] * tbuf[d, s]
        outv[s] = acc
        return carry

    lax.fori_loop(0, _BPW // 16, group_body, 0)
    pltpu.sync_copy(outv, out_hbm.at[pl.ds(base, _BPW)])


def kernel(tuples, node_emb, rela_emb):
    scratch = _harvest(tuples.reshape(-1), node_emb.T, rela_emb.T)
    return _reduce(scratch)


# speedup vs baseline: 15.8162x; 1.4219x over previous
import functools

import jax
import jax.numpy as jnp
from jax import lax
from jax.experimental import pallas as pl
from jax.experimental.pallas import tpu as pltpu
from jax.experimental.pallas import tpu_sc as plsc

_BATCH = 16384
_DIM = 32
_K = 3 * _BATCH
_NW = 32
_BPW = _BATCH // _NW
_WL = 512
_BLKS_PER_TILE = 61
_NWIN = 62
_TAIL_BASE = 999936
_CAP = 8192
_STAGE = 1024
_SCRATCH = _DIM * _K
_DUMP = _SCRATCH
_SCRATCH_PAD = _SCRATCH + 16

_mesh = plsc.VectorSubcoreMesh(core_axis_name="c", subcore_axis_name="s")
_IOTA = lambda: lax.iota(jnp.int32, 16)


@functools.partial(
    pl.kernel,
    mesh=_mesh,
    out_type=jax.ShapeDtypeStruct((_SCRATCH_PAD,), jnp.float32),
    compiler_params=pltpu.CompilerParams(
        needs_layout_passes=False, use_tc_tiling_on_sc=True),
    scratch_types=[
        pltpu.VMEM((1024,), jnp.int32),
        pltpu.VMEM((1024,), jnp.int32),
        pltpu.VMEM((_CAP + 16,), jnp.int32),
        pltpu.VMEM((_CAP + 16,), jnp.int32),
        pltpu.VMEM((_CAP + 16,), jnp.int32),
        pltpu.VMEM((_CAP + 16,), jnp.int32),
        pltpu.VMEM((_DIM, _WL), jnp.float32),
        pltpu.VMEM((_DIM, _WL), jnp.float32),
        pltpu.VMEM((_DIM, _WL), jnp.float32),
        pltpu.VMEM((_DIM, _WL), jnp.float32),
        pltpu.VMEM((_DIM, 64), jnp.float32),
        pltpu.VMEM((_DIM, 64), jnp.float32),
        pltpu.VMEM((16,), jnp.int32),
        pltpu.VMEM((16,), jnp.int32),
        pltpu.VMEM((_STAGE,), jnp.float32),
        pltpu.VMEM((_STAGE,), jnp.int32),
        pltpu.SemaphoreType.DMA,
        pltpu.SemaphoreType.DMA,
        pltpu.SemaphoreType.DMA,
        pltpu.SemaphoreType.DMA,
        pltpu.SemaphoreType.DMA,
    ],
)
def _harvest(tuples_hbm, nodeT_hbm, relaT_hbm, scratch_hbm,
             cbuf0, cbuf1, hNi, hNk, hRi, hRk,
             nwin0, nwin1, rwin0, rwin1, ntail, rtail,
             clane, cslot, stv, std, semi0, semi1, semw0, semw1, semf):
    wid = lax.axis_index("s") * 2 + lax.axis_index("c")
    blk0 = wid * _BLKS_PER_TILE
    lane_lo = blk0 * _WL
    is_last = wid == _NW - 1
    lane_hi = jnp.where(is_last, 1000000, lane_lo + _BLKS_PER_TILE * _WL)
    iota = _IOTA()
    dump_vec = jnp.full((16,), _DUMP, jnp.int32)

    for s in range(_STAGE // 16):
        std[pl.ds(s * 16, 16)] = dump_vec

    cbufs = (cbuf0, cbuf1)
    nwins = (nwin0, nwin1)
    rwins = (rwin0, rwin1)
    semis = (semi0, semi1)
    semws = (semw0, semw1)
    NCHUNK = _K // 1024

    def scan_round(rlo):
        pltpu.async_copy(tuples_hbm.at[pl.ds(0, 1024)], cbufs[0], semis[0])
        pltpu.async_copy(tuples_hbm.at[pl.ds(1024, 1024)], cbufs[1], semis[1])

        def pair_body(p, carry):
            cN, cR = carry
            for b in range(2):
                c = 2 * p + b
                pltpu.make_async_copy(
                    tuples_hbm.at[pl.ds(0, 1024)], cbufs[b], semis[b]).wait()

                def vreg_body(v, carry2, b=b):
                    cN2, cR2 = carry2
                    iv = cbufs[b][pl.ds(v * 16, 16)]
                    kvec = c * 1024 + v * 16 + iota
                    inr = (iv >= lane_lo) & (iv < lane_hi)
                    mN = inr & (kvec < 2 * _BATCH)
                    mR = inr & (kvec >= 2 * _BATCH)

                    def emit(mask, cnt, hi_ref, hk_ref):
                        mi = mask.astype(jnp.int32)
                        csum = lax.cumsum(mi, axis=0)
                        rank = cnt + csum - 1
                        ok = mask & (rank >= rlo) & (rank < rlo + _CAP)
                        pos = jnp.clip(cnt - rlo, 0, _CAP)
                        plsc.store_compressed(hi_ref.at[pl.ds(pos, 16)], iv,
                                              mask=ok)
                        plsc.store_compressed(hk_ref.at[pl.ds(pos, 16)],
                                              kvec, mask=ok)
                        return cnt + csum[15]

                    cN2 = emit(mN, cN2, hNi, hNk)
                    cR2 = emit(mR, cR2, hRi, hRk)
                    return cN2, cR2

                cN, cR = lax.fori_loop(0, 64, vreg_body, (cN, cR))

                def refire(c=c, b=b):
                    off = pl.multiple_of((c + 2) * 1024, 1024)
                    pltpu.async_copy(
                        tuples_hbm.at[pl.ds(off, 1024)], cbufs[b], semis[b])

                pl.when(c + 2 < NCHUNK)(refire)
            return cN, cR

        return lax.fori_loop(0, NCHUNK // 2, pair_body,
                             (jnp.int32(0), jnp.int32(0)))

    def harvest_win(buf, width, wbase, nhits, hi_ref, hk_ref, c2):
        nv = (nhits + 15) // 16

        def vreg_body(v, c2):
            hv = hi_ref[pl.ds(v * 16, 16)]
            hk = hk_ref[pl.ds(v * 16, 16)]
            lanev = hv - wbase
            inwin = (hv >= wbase) & (hv < wbase + width)
            inwin = inwin & ((v * 16 + iota) < nhits)

            def do_vreg(c2):
                m = jnp.sum(inwin.astype(jnp.int32))
                plsc.store_compressed(clane.at[pl.ds(0, 16)], lanev,
                                      mask=inwin)
                plsc.store_compressed(cslot.at[pl.ds(0, 16)], hk, mask=inwin)

                def hit_body(j, c2):
                    jv = jnp.full((16,), j, jnp.int32)
                    colv = plsc.load_gather(clane, [jv])
                    kv = plsc.load_gather(cslot, [jv])
                    rows_lo = iota
                    rows_hi = iota + 16
                    v_lo = plsc.load_gather(buf, [rows_lo, colv])
                    v_hi = plsc.load_gather(buf, [rows_hi, colv])
                    d_lo = iota * _K + kv
                    d_hi = (iota + 16) * _K + kv
                    stv[pl.ds(c2, 16)] = v_lo
                    std[pl.ds(c2, 16)] = d_lo
                    stv[pl.ds(c2 + 16, 16)] = v_hi
                    std[pl.ds(c2 + 16, 16)] = d_hi
                    c2n = c2 + 32

                    def flush():
                        pltpu.async_copy(
                            stv, scratch_hbm.at[std], semf).wait()
                        for s in range(_STAGE // 16):
                            std[pl.ds(s * 16, 16)] = dump_vec

                    do_flush = c2n > _STAGE - 32
                    pl.when(do_flush)(flush)
                    return jnp.where(do_flush, 0, c2n)

                return c2 + 0 * m

            any_hit = jnp.any(inwin)
            return lax.cond(any_hit, do_vreg, lambda c2: c2, c2)

        return lax.fori_loop(0, nv, vreg_body, c2)

    def round_body(carry):
        r, c2, doneN, doneR, totN, totR = carry
        rlo = r * _CAP
        totN, totR = scan_round(rlo)
        nN = jnp.clip(totN - rlo, 0, _CAP)
        nR = jnp.clip(totR - rlo, 0, _CAP)

        off0 = pl.multiple_of(blk0 * _WL, _WL)
        off1 = pl.multiple_of((blk0 + 1) * _WL, _WL)
        pltpu.async_copy(nodeT_hbm.at[:, pl.ds(off0, _WL)], nwins[0], semws[0])
        pltpu.async_copy(relaT_hbm.at[:, pl.ds(off0, _WL)], rwins[0], semws[0])
        pltpu.async_copy(nodeT_hbm.at[:, pl.ds(off1, _WL)], nwins[1], semws[1])
        pltpu.async_copy(relaT_hbm.at[:, pl.ds(off1, _WL)], rwins[1], semws[1])

        def win_pair(p, c2):
            for b in range(2):
                w = 2 * p + b
                pltpu.make_async_copy(
                    nodeT_hbm.at[:, pl.ds(0, _WL)], nwins[b],
                    semws[b]).wait()
                pltpu.make_async_copy(
                    relaT_hbm.at[:, pl.ds(0, _WL)], rwins[b],
                    semws[b]).wait()
                wbase = (blk0 + w) * _WL
                c2 = harvest_win(nwins[b], _WL, wbase, nN, hNi, hNk, c2)
                c2 = harvest_win(rwins[b], _WL, wbase, nR, hRi, hRk, c2)

                def refire(w=w, b=b):
                    off = pl.multiple_of((blk0 + w + 2) * _WL, _WL)
                    pltpu.async_copy(nodeT_hbm.at[:, pl.ds(off, _WL)],
                                     nwins[b], semws[b])
                    pltpu.async_copy(relaT_hbm.at[:, pl.ds(off, _WL)],
                                     rwins[b], semws[b])

                pl.when(w + 2 < _NWIN)(refire)
            return c2

        c2 = lax.fori_loop(0, _NWIN // 2, win_pair, c2)

        def tail(c2):
            pltpu.async_copy(
                nodeT_hbm.at[:, pl.ds(_TAIL_BASE, 64)], ntail, semws[0]).wait()
            pltpu.async_copy(
                relaT_hbm.at[:, pl.ds(_TAIL_BASE, 64)], rtail, semws[0]).wait()
            c2 = harvest_win(ntail, 64, _TAIL_BASE, nN, hNi, hNk, c2)
            c2 = harvest_win(rtail, 64, _TAIL_BASE, nR, hRi, hRk, c2)
            return c2

        c2 = lax.cond(is_last, tail, lambda c2: c2, c2)
        doneN = jnp.minimum(rlo + _CAP, totN)
        doneR = jnp.minimum(rlo + _CAP, totR)
        return r + 1, c2, doneN, doneR, totN, totR

    def round_cond(carry):
        r, c2, doneN, doneR, totN, totR = carry
        return (r == 0) | (doneN < totN) | (doneR < totR)

    init = (jnp.int32(0), jnp.int32(0), jnp.int32(0), jnp.int32(0),
            jnp.int32(-1), jnp.int32(-1))
    _, c2, _, _, _, _ = lax.while_loop(round_cond, round_body, init)

    def final_flush():
        pltpu.async_copy(stv, scratch_hbm.at[std], semf).wait()

    pl.when(c2 > 0)(final_flush)


@functools.partial(
    pl.kernel,
    mesh=_mesh,
    out_type=jax.ShapeDtypeStruct((_BATCH,), jnp.float32),
    compiler_params=pltpu.CompilerParams(
        needs_layout_passes=False, use_tc_tiling_on_sc=True),
    scratch_types=[
        pltpu.VMEM((_DIM, _BPW), jnp.float32),
        pltpu.VMEM((_DIM, _BPW), jnp.float32),
        pltpu.VMEM((_DIM, _BPW), jnp.float32),
        pltpu.VMEM((_BPW,), jnp.float32),
        pltpu.SemaphoreType.DMA,
    ],
)
def _reduce(scratch_hbm, out_hbm, hbuf, tbuf, rbuf, outv, sem):
    wid = lax.axis_index("s") * 2 + lax.axis_index("c")
    base = wid * _BPW
    copies = []
    for d in range(_DIM):
        copies.append(pltpu.async_copy(
            scratch_hbm.at[pl.ds(d * _K + base, _BPW)], hbuf.at[d], sem))
        copies.append(pltpu.async_copy(
            scratch_hbm.at[pl.ds(d * _K + _BATCH + base, _BPW)],
            tbuf.at[d], sem))
        copies.append(pltpu.async_copy(
            scratch_hbm.at[pl.ds(d * _K + 2 * _BATCH + base, _BPW)],
            rbuf.at[d], sem))
    for c in copies:
        c.wait()

    def group_body(g, carry):
        s = pl.ds(g * 16, 16)
        acc = jnp.zeros((16,), jnp.float32)
        for d in range(_DIM):
            acc = acc + hbuf[d, s] * rbuf[d, s] * tbuf[d, s]
        outv[s] = acc
        return carry

    lax.fori_loop(0, _BPW // 16, group_body, 0)
    pltpu.sync_copy(outv, out_hbm.at[pl.ds(base, _BPW)])


def kernel(tuples, node_emb, rela_emb):
    scratch = _harvest(tuples.reshape(-1), node_emb.T, rela_emb.T)
    return _reduce(scratch)
